# Initial kernel scaffold; baseline (speedup 1.0000x reference)
import jax
import jax.numpy as jnp
from jax.experimental import pallas as pl


def kernel(weights, importance_mask):
    raise NotImplementedError("write your pallas kernel here")


# speedup vs baseline: 1.0102x; 1.0102x over previous
import jax
import jax.numpy as jnp
from jax.experimental import pallas as pl
from jax.experimental.pallas import tpu as pltpu

_SCALE = 0.95
_THRESH = 0.01

_N = 4096
_BLOCK_ROWS = 256
_GRID = _N // _BLOCK_ROWS


def _body(w_ref, m_ref, out_ref, cnt_ref):
    s = w_ref[...] * _SCALE
    prune = jnp.logical_and(m_ref[...] == 0, jnp.abs(s) < _THRESH)
    out_ref[...] = jnp.where(prune, 0.0, s)
    c = jnp.sum(prune.astype(jnp.int32))
    cnt_ref[0, 0] = jnp.where(pl.program_id(0) == 0, c, cnt_ref[0, 0] + c)


def kernel(weights, importance_mask):
    mask_i8 = importance_mask.view(jnp.int8)
    out, cnt = pl.pallas_call(
        _body,
        grid=(_GRID,),
        in_specs=[
            pl.BlockSpec((_BLOCK_ROWS, _N), lambda i: (i, 0)),
            pl.BlockSpec((_BLOCK_ROWS, _N), lambda i: (i, 0)),
        ],
        out_specs=[
            pl.BlockSpec((_BLOCK_ROWS, _N), lambda i: (i, 0)),
            pl.BlockSpec(memory_space=pltpu.SMEM, block_shape=(1, 1), index_map=lambda i: (0, 0)),
        ],
        out_shape=[
            jax.ShapeDtypeStruct((_N, _N), jnp.float32),
            jax.ShapeDtypeStruct((1, 1), jnp.int32),
        ],
    )(weights, mask_i8)
    return out, cnt[0, 0]
